# baseline (speedup 1.0000x reference)
import jax
import jax.numpy as jnp
import numpy as np
from jax import lax
from jax.experimental import pallas as pl
from jax.experimental.pallas import tpu as pltpu

_B = 16
_HW = 147456
_K_SEL = 16384
_NUM_MATCHES = 1024.0
_HBLK = 9216
_NB = _HW // _HBLK
_CW = 9216
_NC = _HW // _CW
_SUBW = 1152
_NSUB = _HW // _SUBW
_NEG = -3.0e38
_NEG_BITS = int(np.float32(_NEG).view(np.int32))
_INT_MIN = -2147483648


def _body(la_ref, lb_ref, da_ref, db_ref, m_ref, out_ref,
          t_ref, kd_ref, las_ref, smp_ref, mva_ref, mvb_ref):
    i = pl.program_id(0)

    @pl.when(i == 0)
    def _init():
        mva_ref[...] = jnp.full((_B, _SUBW), _NEG, jnp.float32)
        mvb_ref[...] = jnp.full((_B, _SUBW), _NEG, jnp.float32)

    mva = mva_ref[...]
    mvb = mvb_ref[...]
    for j in range(_HBLK // _SUBW):
        sl = slice(j * _SUBW, (j + 1) * _SUBW)
        gsl = pl.ds(i * _HBLK + j * _SUBW, _SUBW)
        laj = la_ref[:, sl]
        lbj = lb_ref[:, sl]
        mj = m_ref[:, sl] > 0
        las_ref[:, gsl] = laj
        t_ref[:, gsl] = laj + lbj
        dj = jnp.where(mj, (da_ref[:, sl] + db_ref[:, sl]) * 50.0, _NEG)
        kd_ref[:, gsl] = lax.bitcast_convert_type(dj, jnp.int32)
        mva = jnp.maximum(mva, jnp.where(mj, laj, _NEG))
        mvb = jnp.maximum(mvb, jnp.where(mj, lbj, _NEG))
    mva_ref[...] = mva
    mvb_ref[...] = mvb

    @pl.when(i == _NB - 1)
    def _finish():
        m_a = jnp.max(mva_ref[...], axis=1, keepdims=True)
        m_b = jnp.max(mvb_ref[...], axis=1, keepdims=True)

        def _se(c, carry):
            aa, ab = carry
            sl = pl.ds(c * _SUBW, _SUBW)
            valid = kd_ref[:, sl] != _NEG_BITS
            laj = las_ref[:, sl]
            aa = aa + jnp.where(valid, jnp.exp(laj - m_a), 0.0)
            ab = ab + jnp.where(valid,
                                jnp.exp((t_ref[:, sl] - laj) - m_b), 0.0)
            return aa, ab
        z = jnp.zeros((_B, _SUBW), jnp.float32)
        acc_a, acc_b = lax.fori_loop(0, _NSUB, _se, (z, z))
        s_a = jnp.sum(acc_a, axis=1, keepdims=True)
        s_b = jnp.sum(acc_b, axis=1, keepdims=True)

        lse = m_a + jnp.log(s_a) + m_b + jnp.log(s_b)

        def _fwd(ii):
            return ii ^ (lax.shift_right_arithmetic(ii, 31)
                         & jnp.int32(0x7FFFFFFF))

        def _smp(c, carry):
            sl = pl.ds(c * _CW, 128)
            dd = lax.bitcast_convert_type(kd_ref[:, sl], jnp.float32)
            s = t_ref[:, sl] - lse + dd
            smp_ref[:, pl.ds(c * 128, 128)] = _fwd(
                lax.bitcast_convert_type(s, jnp.int32))
            return carry
        lax.fori_loop(0, _NC, _smp, jnp.int32(0))

        def _count_ge(cand):
            def _cb(c, acc):
                base = c * (2 * _SUBW)
                k0 = kd_ref[:, pl.ds(base, _SUBW)]
                k1 = kd_ref[:, pl.ds(base + _SUBW, _SUBW)]
                return acc + (jnp.where(k0 >= cand, 1.0, 0.0)
                              + jnp.where(k1 >= cand, 1.0, 0.0))
            acc = lax.fori_loop(0, _NSUB // 2, _cb,
                                jnp.zeros((_B, _SUBW), jnp.float32))
            return jnp.sum(acc)

        ss = smp_ref[...]

        def _sel_sample(k):
            cntp = jnp.sum(jnp.where(ss >= 0, 1.0, 0.0))
            pref0 = jnp.where(cntp >= k, 0, _INT_MIN).astype(jnp.int32)

            def _bit(j, p):
                cand = p | lax.shift_left(jnp.int32(1), jnp.int32(30) - j)
                c = jnp.sum(jnp.where(ss >= cand, 1.0, 0.0))
                return jnp.where(c >= k, cand, p)
            return lax.fori_loop(0, 31, _bit, pref0)

        t_hi = _sel_sample(122.0)
        t_lo = _sel_sample(333.0)

        def _mkc(c, carry):
            alo, ahi = carry
            sl = pl.ds(c * _SUBW, _SUBW)
            dd = lax.bitcast_convert_type(kd_ref[:, sl], jnp.float32)
            s = t_ref[:, sl] - lse + dd
            key = _fwd(lax.bitcast_convert_type(s, jnp.int32))
            kd_ref[:, sl] = key
            alo = alo + jnp.where(key >= t_lo, 1.0, 0.0)
            ahi = ahi + jnp.where(key >= t_hi, 1.0, 0.0)
            return alo, ahi
        z2 = jnp.zeros((_B, _SUBW), jnp.float32)
        acc_lo, acc_hi = lax.fori_loop(0, _NSUB, _mkc, (z2, z2))
        c_lo = jnp.sum(acc_lo)
        c_hi = jnp.sum(acc_hi)

        def _inv(k):
            return k ^ (lax.shift_right_arithmetic(k, 31)
                        & jnp.int32(0x7FFFFFFF))

        def _val(k):
            return lax.bitcast_convert_type(_inv(k), jnp.float32)

        def _ux(k):
            return k ^ _INT_MIN

        def _udiff(lo, hi):
            return _ux(hi) - _ux(lo)

        kf = jnp.float32(_K_SEL)
        lo0 = jnp.int32(_INT_MIN)
        hi0 = jnp.int32(2147483647)
        clo0 = jnp.float32(_B * _HW)
        chi0 = jnp.float32(0.0)
        sel1 = c_lo >= kf
        lo0 = jnp.where(sel1, t_lo, lo0)
        clo0 = jnp.where(sel1, c_lo, clo0)
        hi0 = jnp.where(sel1, hi0, t_lo)
        chi0 = jnp.where(sel1, chi0, c_lo)
        upd_lo = (c_hi >= kf) & (t_hi > lo0)
        upd_hi = (c_hi < kf) & (t_hi < hi0)
        lo0 = jnp.where(upd_lo, t_hi, lo0)
        clo0 = jnp.where(upd_lo, c_hi, clo0)
        hi0 = jnp.where(upd_hi, t_hi, hi0)
        chi0 = jnp.where(upd_hi, c_hi, chi0)

        def _icond(carry):
            it, lo, hi, _, _ = carry
            return (it < 6) & (lax.shift_right_logical(_udiff(lo, hi), 1)
                               > 0)

        def _istep(carry):
            it, lo, hi, clo, chi = carry
            vlo = _val(lo)
            vhi = _val(hi)
            frac = (clo - kf) / jnp.maximum(clo - chi, 1.0)
            vm = vlo + (vhi - vlo) * frac
            km = lax.bitcast_convert_type(vm, jnp.int32)
            km = _inv(km)
            km = _ux(jnp.minimum(jnp.maximum(_ux(km), _ux(lo) + 1),
                                 _ux(hi) - 1))
            cm = _count_ge(km)
            s = cm >= kf
            lo = jnp.where(s, km, lo)
            clo = jnp.where(s, cm, clo)
            hi = jnp.where(s, hi, km)
            chi = jnp.where(s, chi, cm)
            return it + 1, lo, hi, clo, chi

        _, lo0, hi0, _, _ = lax.while_loop(
            _icond, _istep, (jnp.int32(0), lo0, hi0, clo0, chi0))

        def _cond(carry):
            lo, hi = carry
            return lax.shift_right_logical(_udiff(lo, hi), 1) > 0

        def _step(carry):
            lo, hi = carry
            half = lax.shift_right_logical(_udiff(lo, hi), 1)
            mid = _ux(_ux(lo) + half)
            cnt = _count_ge(mid)
            lo = jnp.where(cnt >= kf, mid, lo)
            hi = jnp.where(cnt >= kf, hi, mid)
            return lo, hi

        kth, _ = lax.while_loop(_cond, _step, (lo0, hi0))

        def _sb(c, acc):
            sl = pl.ds(c * _SUBW, _SUBW)
            kk = kd_ref[:, sl]
            g = t_ref[:, sl] - lse
            return acc + jnp.where(kk > kth, g, 0.0)
        gacc = lax.fori_loop(0, _NSUB, _sb,
                             jnp.zeros((_B, _SUBW), jnp.float32))
        out_ref[0, 0] = -jnp.sum(gacc) / _NUM_MATCHES


def kernel(logits_A, logits_B_to_A, detections_A, detections_B_to_A, mask):
    la = logits_A.reshape(_B, _HW)
    lb = logits_B_to_A.reshape(_B, _HW)
    mask_i = mask.astype(jnp.int32)

    blk = lambda: pl.BlockSpec((_B, _HBLK), lambda i: (0, i))
    out = pl.pallas_call(
        _body,
        grid=(_NB,),
        in_specs=[blk(), blk(), blk(), blk(), blk()],
        out_specs=pl.BlockSpec(memory_space=pltpu.SMEM),
        out_shape=jax.ShapeDtypeStruct((1, 1), jnp.float32),
        scratch_shapes=[
            pltpu.VMEM((_B, _HW), jnp.float32),
            pltpu.VMEM((_B, _HW), jnp.int32),
            pltpu.VMEM((_B, _HW), jnp.float32),
            pltpu.VMEM((_B, 16 * 128), jnp.int32),
            pltpu.VMEM((_B, _SUBW), jnp.float32),
            pltpu.VMEM((_B, _SUBW), jnp.float32),
        ],
    )(la, lb, detections_A, detections_B_to_A, mask_i)
    return out[0, 0]


# speedup vs baseline: 1.0565x; 1.0565x over previous
import jax
import jax.numpy as jnp
import numpy as np
from jax import lax
from jax.experimental import pallas as pl
from jax.experimental.pallas import tpu as pltpu

_B = 16
_HW = 147456
_K_SEL = 16384
_NUM_MATCHES = 1024.0
_HBLK = 9216
_NB = _HW // _HBLK
_CW = 9216
_NC = _HW // _CW
_SUBW = 1152
_NSUB = _HW // _SUBW
_NEG = -3.0e38
_NEG_BITS = int(np.float32(_NEG).view(np.int32))
_INT_MIN = -2147483648


def _body(la_ref, lb_ref, da_ref, db_ref, m_ref, out_ref,
          t_ref, kd_ref, las_ref, smp_ref, mva_ref, mvb_ref):
    i = pl.program_id(0)

    @pl.when(i == 0)
    def _init():
        mva_ref[...] = jnp.full((_B, _SUBW), _NEG, jnp.float32)
        mvb_ref[...] = jnp.full((_B, _SUBW), _NEG, jnp.float32)

    mva = mva_ref[...]
    mvb = mvb_ref[...]
    for j in range(_HBLK // _SUBW):
        sl = slice(j * _SUBW, (j + 1) * _SUBW)
        gsl = pl.ds(i * _HBLK + j * _SUBW, _SUBW)
        laj = la_ref[:, sl]
        lbj = lb_ref[:, sl]
        mj = m_ref[:, sl] > 0
        las_ref[:, gsl] = laj
        t_ref[:, gsl] = laj + lbj
        dj = jnp.where(mj, (da_ref[:, sl] + db_ref[:, sl]) * 50.0, _NEG)
        kd_ref[:, gsl] = lax.bitcast_convert_type(dj, jnp.int32)
        mva = jnp.maximum(mva, jnp.where(mj, laj, _NEG))
        mvb = jnp.maximum(mvb, jnp.where(mj, lbj, _NEG))
    mva_ref[...] = mva
    mvb_ref[...] = mvb

    @pl.when(i == _NB - 1)
    def _finish():
        m_a = jnp.max(mva_ref[...], axis=1, keepdims=True)
        m_b = jnp.max(mvb_ref[...], axis=1, keepdims=True)

        def _se(c, carry):
            aa, ab = carry
            sl = pl.ds(c * _SUBW, _SUBW)
            valid = kd_ref[:, sl] != _NEG_BITS
            laj = las_ref[:, sl]
            aa = aa + jnp.where(valid, jnp.exp(laj - m_a), 0.0)
            ab = ab + jnp.where(valid,
                                jnp.exp((t_ref[:, sl] - laj) - m_b), 0.0)
            return aa, ab
        z = jnp.zeros((_B, _SUBW), jnp.float32)
        acc_a, acc_b = lax.fori_loop(0, _NSUB, _se, (z, z))
        s_a = jnp.sum(acc_a, axis=1, keepdims=True)
        s_b = jnp.sum(acc_b, axis=1, keepdims=True)

        lse = m_a + jnp.log(s_a) + m_b + jnp.log(s_b)

        def _fwd(ii):
            return ii ^ (lax.shift_right_arithmetic(ii, 31)
                         & jnp.int32(0x7FFFFFFF))

        def _smp(c, carry):
            sl = pl.ds(c * _CW, 128)
            dd = lax.bitcast_convert_type(kd_ref[:, sl], jnp.float32)
            s = t_ref[:, sl] - lse + dd
            smp_ref[:, pl.ds(c * 128, 128)] = _fwd(
                lax.bitcast_convert_type(s, jnp.int32))
            return carry
        lax.fori_loop(0, _NC, _smp, jnp.int32(0))

        def _count_ge(cand):
            def _cb(c, acc):
                base = c * (2 * _SUBW)
                k0 = kd_ref[:, pl.ds(base, _SUBW)]
                k1 = kd_ref[:, pl.ds(base + _SUBW, _SUBW)]
                return acc + (jnp.where(k0 >= cand, 1.0, 0.0)
                              + jnp.where(k1 >= cand, 1.0, 0.0))
            acc = lax.fori_loop(0, _NSUB // 2, _cb,
                                jnp.zeros((_B, _SUBW), jnp.float32))
            return jnp.sum(acc)

        ss = smp_ref[...]

        ka = jnp.float32(122.0)
        kb = jnp.float32(333.0)
        cntp = jnp.sum(jnp.where(ss >= 0, 1.0, 0.0))
        pa0 = jnp.where(cntp >= ka, 0, _INT_MIN).astype(jnp.int32)
        pb0 = jnp.where(cntp >= kb, 0, _INT_MIN).astype(jnp.int32)

        def _bit(j, carry):
            pa, pb = carry
            bit = lax.shift_left(jnp.int32(1), jnp.int32(30) - j)
            ca_c = pa | bit
            cb_c = pb | bit
            ca = jnp.sum(jnp.where(ss >= ca_c, 1.0, 0.0))
            cb = jnp.sum(jnp.where(ss >= cb_c, 1.0, 0.0))
            return (jnp.where(ca >= ka, ca_c, pa),
                    jnp.where(cb >= kb, cb_c, pb))
        t_hi, t_lo = lax.fori_loop(0, 31, _bit, (pa0, pb0))

        def _mkc(c, carry):
            alo, ahi = carry
            sl = pl.ds(c * _SUBW, _SUBW)
            dd = lax.bitcast_convert_type(kd_ref[:, sl], jnp.float32)
            s = t_ref[:, sl] - lse + dd
            key = _fwd(lax.bitcast_convert_type(s, jnp.int32))
            kd_ref[:, sl] = key
            alo = alo + jnp.where(key >= t_lo, 1.0, 0.0)
            ahi = ahi + jnp.where(key >= t_hi, 1.0, 0.0)
            return alo, ahi
        z2 = jnp.zeros((_B, _SUBW), jnp.float32)
        acc_lo, acc_hi = lax.fori_loop(0, _NSUB, _mkc, (z2, z2))
        c_lo = jnp.sum(acc_lo)
        c_hi = jnp.sum(acc_hi)

        def _inv(k):
            return k ^ (lax.shift_right_arithmetic(k, 31)
                        & jnp.int32(0x7FFFFFFF))

        def _val(k):
            return lax.bitcast_convert_type(_inv(k), jnp.float32)

        def _ux(k):
            return k ^ _INT_MIN

        def _udiff(lo, hi):
            return _ux(hi) - _ux(lo)

        kf = jnp.float32(_K_SEL)
        lo0 = jnp.int32(_INT_MIN)
        hi0 = jnp.int32(2147483647)
        clo0 = jnp.float32(_B * _HW)
        chi0 = jnp.float32(0.0)
        sel1 = c_lo >= kf
        lo0 = jnp.where(sel1, t_lo, lo0)
        clo0 = jnp.where(sel1, c_lo, clo0)
        hi0 = jnp.where(sel1, hi0, t_lo)
        chi0 = jnp.where(sel1, chi0, c_lo)
        upd_lo = (c_hi >= kf) & (t_hi > lo0)
        upd_hi = (c_hi < kf) & (t_hi < hi0)
        lo0 = jnp.where(upd_lo, t_hi, lo0)
        clo0 = jnp.where(upd_lo, c_hi, clo0)
        hi0 = jnp.where(upd_hi, t_hi, hi0)
        chi0 = jnp.where(upd_hi, c_hi, chi0)

        def _icond(carry):
            it, lo, hi, _, _ = carry
            return (it < 6) & (lax.shift_right_logical(_udiff(lo, hi), 1)
                               > 0)

        def _istep(carry):
            it, lo, hi, clo, chi = carry
            vlo = _val(lo)
            vhi = _val(hi)
            frac = (clo - kf) / jnp.maximum(clo - chi, 1.0)
            vm = vlo + (vhi - vlo) * frac
            km = lax.bitcast_convert_type(vm, jnp.int32)
            km = _inv(km)
            km = _ux(jnp.minimum(jnp.maximum(_ux(km), _ux(lo) + 1),
                                 _ux(hi) - 1))
            cm = _count_ge(km)
            s = cm >= kf
            lo = jnp.where(s, km, lo)
            clo = jnp.where(s, cm, clo)
            hi = jnp.where(s, hi, km)
            chi = jnp.where(s, chi, cm)
            return it + 1, lo, hi, clo, chi

        _, lo0, hi0, _, _ = lax.while_loop(
            _icond, _istep, (jnp.int32(0), lo0, hi0, clo0, chi0))

        def _cond(carry):
            lo, hi = carry
            return lax.shift_right_logical(_udiff(lo, hi), 1) > 0

        def _step(carry):
            lo, hi = carry
            half = lax.shift_right_logical(_udiff(lo, hi), 1)
            mid = _ux(_ux(lo) + half)
            cnt = _count_ge(mid)
            lo = jnp.where(cnt >= kf, mid, lo)
            hi = jnp.where(cnt >= kf, hi, mid)
            return lo, hi

        kth, _ = lax.while_loop(_cond, _step, (lo0, hi0))

        def _sb(c, acc):
            sl = pl.ds(c * _SUBW, _SUBW)
            kk = kd_ref[:, sl]
            g = t_ref[:, sl] - lse
            return acc + jnp.where(kk > kth, g, 0.0)
        gacc = lax.fori_loop(0, _NSUB, _sb,
                             jnp.zeros((_B, _SUBW), jnp.float32))
        out_ref[0, 0] = -jnp.sum(gacc) / _NUM_MATCHES


def kernel(logits_A, logits_B_to_A, detections_A, detections_B_to_A, mask):
    la = logits_A.reshape(_B, _HW)
    lb = logits_B_to_A.reshape(_B, _HW)
    mask_i = mask.astype(jnp.int32)

    blk = lambda: pl.BlockSpec((_B, _HBLK), lambda i: (0, i))
    out = pl.pallas_call(
        _body,
        grid=(_NB,),
        in_specs=[blk(), blk(), blk(), blk(), blk()],
        out_specs=pl.BlockSpec(memory_space=pltpu.SMEM),
        out_shape=jax.ShapeDtypeStruct((1, 1), jnp.float32),
        scratch_shapes=[
            pltpu.VMEM((_B, _HW), jnp.float32),
            pltpu.VMEM((_B, _HW), jnp.int32),
            pltpu.VMEM((_B, _HW), jnp.float32),
            pltpu.VMEM((_B, 16 * 128), jnp.int32),
            pltpu.VMEM((_B, _SUBW), jnp.float32),
            pltpu.VMEM((_B, _SUBW), jnp.float32),
        ],
    )(la, lb, detections_A, detections_B_to_A, mask_i)
    return out[0, 0]
